# Initial kernel scaffold; baseline (speedup 1.0000x reference)
import jax
import jax.numpy as jnp
from jax.experimental import pallas as pl


def kernel(x, edge_index, w_self1, w_neigh1, b1, w_self2, w_neigh2, b2):
    raise NotImplementedError("write your pallas kernel here")


# speedup vs baseline: 8.7217x; 8.7217x over previous
import functools

import jax
import jax.numpy as jnp
from jax import lax
from jax.experimental import pallas as pl
from jax.experimental.pallas import tpu as pltpu
from jax.experimental.pallas import tpu_sc as plsc

N_NODES = 10000
N_EDGES = 320000
IN_DIM = 128
HIDDEN_DIM = 64
EMBED_DIM = 32

NC = 2
NS = 16
NW = NC * NS
CHUNK = 128
K_CHUNKS = 79
E_PER_W = CHUNK * K_CHUNKS
E_PAD = E_PER_W * NW
SEG_PAD = 10240
RPT = SEG_PAD // NS


def _sc_segment_sum(d_model, with_counts):
  mesh = plsc.VectorSubcoreMesh(
      core_axis_name="c", subcore_axis_name="s", num_cores=NC,
      num_subcores=NS)

  out_type = [jax.ShapeDtypeStruct((NC, SEG_PAD, d_model), jnp.float32)]
  if with_counts:
    out_type.append(jax.ShapeDtypeStruct((NC, SEG_PAD), jnp.float32))

  scratch = dict(
      src_v=pltpu.VMEM((K_CHUNKS, CHUNK), jnp.int32),
      dst_v=pltpu.VMEM((K_CHUNKS, CHUNK), jnp.int32),
      rows_v=pltpu.VMEM((CHUNK, d_model), jnp.float32),
      stage_v=pltpu.VMEM((RPT, d_model), jnp.float32),
      agg_sh=pltpu.VMEM_SHARED((SEG_PAD, d_model), jnp.float32),
  )
  if with_counts:
    scratch.update(
        ones_v=pltpu.VMEM((CHUNK,), jnp.float32),
        cstage_v=pltpu.VMEM((RPT,), jnp.float32),
        cnt_sh=pltpu.VMEM_SHARED((SEG_PAD,), jnp.float32),
    )

  @functools.partial(
      pl.kernel, out_type=tuple(out_type), mesh=mesh,
      scratch_types=scratch,
      compiler_params=pltpu.CompilerParams(use_tc_tiling_on_sc=False),
  )
  def kern(table_hbm, src_hbm, dst_hbm, z_hbm, z1_hbm, *refs, src_v, dst_v,
           rows_v, stage_v, agg_sh, ones_v=None, cstage_v=None, cnt_sh=None):
    if with_counts:
      agg_hbm, cnt_hbm = refs
    else:
      (agg_hbm,) = refs
      del z1_hbm
    c = lax.axis_index("c")
    s = lax.axis_index("s")
    w = s * NC + c

    pltpu.sync_copy(z_hbm, stage_v)
    pltpu.sync_copy(stage_v, agg_sh.at[pl.ds(s * RPT, RPT)])
    if with_counts:
      pltpu.sync_copy(z1_hbm, cstage_v)
      pltpu.sync_copy(cstage_v, cnt_sh.at[pl.ds(s * RPT, RPT)])
      for j in range(CHUNK // 16):
        ones_v[pl.ds(j * 16, 16)] = jnp.ones((16,), jnp.float32)
    plsc.subcore_barrier()

    pltpu.sync_copy(src_hbm.at[w], src_v)
    pltpu.sync_copy(dst_hbm.at[w], dst_v)

    def body(j, carry):
      pltpu.sync_copy(table_hbm.at[src_v.at[j]], rows_v)
      pltpu.sync_copy(rows_v, agg_sh.at[dst_v.at[j]], add=True)
      if with_counts:
        pltpu.sync_copy(ones_v, cnt_sh.at[dst_v.at[j]], add=True)
      return carry

    lax.fori_loop(0, K_CHUNKS, body, 0)
    plsc.subcore_barrier()

    pltpu.sync_copy(agg_sh.at[pl.ds(s * RPT, RPT)], stage_v)
    pltpu.sync_copy(stage_v, agg_hbm.at[c, pl.ds(s * RPT, RPT)])
    if with_counts:
      pltpu.sync_copy(cnt_sh.at[pl.ds(s * RPT, RPT)], cstage_v)
      pltpu.sync_copy(cstage_v, cnt_hbm.at[c, pl.ds(s * RPT, RPT)])

  return kern


def _tc_layer1(x, w_self1, w_neigh1, b1):
  def body(x_ref, ws_ref, wn_ref, b_ref, s1_ref, xp_ref):
    xv = x_ref[...]
    s1_ref[...] = (
        jnp.dot(xv, ws_ref[...], preferred_element_type=jnp.float32)
        + b_ref[...]
    )
    xp_ref[...] = jnp.dot(xv, wn_ref[...], preferred_element_type=jnp.float32)

  return pl.pallas_call(
      body,
      out_shape=(
          jax.ShapeDtypeStruct((N_NODES, HIDDEN_DIM), jnp.float32),
          jax.ShapeDtypeStruct((N_NODES, HIDDEN_DIM), jnp.float32),
      ),
  )(x, w_self1, w_neigh1, b1.reshape(1, HIDDEN_DIM))


def _tc_layer2(s1, p, cnt, w_self2, w_neigh2, b2):
  def body(s1_ref, p_ref, c_ref, ws_ref, wn_ref, b_ref, s2_ref, hp_ref,
           r_ref):
    psum = p_ref[0, :N_NODES, :] + p_ref[1, :N_NODES, :]
    csum = c_ref[0, :N_NODES, :] + c_ref[1, :N_NODES, :]
    recip = 1.0 / jnp.maximum(csum, 1.0)
    r_ref[...] = recip
    h = jnp.maximum(s1_ref[...] + psum * recip, 0.0)
    s2_ref[...] = (
        jnp.dot(h, ws_ref[...], preferred_element_type=jnp.float32)
        + b_ref[...]
    )
    hp_ref[...] = jnp.dot(h, wn_ref[...], preferred_element_type=jnp.float32)

  return pl.pallas_call(
      body,
      out_shape=(
          jax.ShapeDtypeStruct((N_NODES, EMBED_DIM), jnp.float32),
          jax.ShapeDtypeStruct((N_NODES, EMBED_DIM), jnp.float32),
          jax.ShapeDtypeStruct((N_NODES, 1), jnp.float32),
      ),
  )(s1, p, cnt, w_self2, w_neigh2, b2.reshape(1, EMBED_DIM))


def _tc_final(s2, q, recip):
  def body(s2_ref, q_ref, r_ref, out_ref):
    qsum = q_ref[0, :N_NODES, :] + q_ref[1, :N_NODES, :]
    out_ref[...] = s2_ref[...] + qsum * r_ref[...]

  return pl.pallas_call(
      body,
      out_shape=jax.ShapeDtypeStruct((N_NODES, EMBED_DIM), jnp.float32),
  )(s2, q, recip)


@jax.jit
def kernel(x, edge_index, w_self1, w_neigh1, b1, w_self2, w_neigh2, b2):
  src = edge_index[0].astype(jnp.int32)
  dst = edge_index[1].astype(jnp.int32)
  pad = E_PAD - N_EDGES
  src3 = jnp.concatenate([src, jnp.zeros((pad,), jnp.int32)]).reshape(
      NW, K_CHUNKS, CHUNK)
  dst3 = jnp.concatenate(
      [dst, jnp.full((pad,), N_NODES, jnp.int32)]).reshape(
          NW, K_CHUNKS, CHUNK)
  z64 = jnp.zeros((RPT, HIDDEN_DIM), jnp.float32)
  z32 = jnp.zeros((RPT, EMBED_DIM), jnp.float32)
  z1 = jnp.zeros((RPT,), jnp.float32)

  s1, xp = _tc_layer1(x, w_self1, w_neigh1, b1)
  p, cnt = _sc_segment_sum(HIDDEN_DIM, True)(xp, src3, dst3, z64, z1)
  s2, hp, recip = _tc_layer2(s1, p, cnt.reshape(NC, SEG_PAD, 1),
                             w_self2, w_neigh2, b2)
  (q,) = _sc_segment_sum(EMBED_DIM, False)(hp, src3, dst3, z32, z1)
  return _tc_final(s2, q, recip)
